# baseline (speedup 1.0000x reference)
import jax
import jax.numpy as jnp
from jax import lax
from jax.experimental import pallas as pl
from jax.experimental.pallas import tpu as pltpu
from jax.experimental.pallas import tpu_sc as plsc

VOCAB = 1000000
EMBED_DIM = 32
BATCH = 16384
HIST = 50

NUM_CORES = 2
NUM_SUBCORES = 16
NUM_WORKERS = NUM_CORES * NUM_SUBCORES

B_PER_W = BATCH // NUM_WORKERS
IDX_PER_W = B_PER_W * HIST
LANES = 16
KBLOCKS = B_PER_W // LANES


HALF = B_PER_W // 2
KB_HALF = KBLOCKS // 2


def _body(idx_hbm, table_hbm, out_hbm, idx_raw,
          gidx_a0, gidx_a1, gidx_b0, gidx_b1,
          rb_a0, rb_a1, rb_b0, rb_b1, tb_a, tb_b,
          gsem_a, gsem_b, osem_a, osem_b):
    wid = lax.axis_index("s") * NUM_CORES + lax.axis_index("c")
    b0 = wid * B_PER_W

    pltpu.sync_copy(idx_hbm.at[pl.ds(b0 * HIST, IDX_PER_W)], idx_raw)

    iota = lax.iota(jnp.int32, LANES)
    iota_h = iota * HIST

    def regroup(h, gidx0, gidx1):
        for kb in range(KBLOCKS):
            v = plsc.load_gather(idx_raw, [iota_h + (kb * LANES * HIST + h)])
            if kb < KB_HALF:
                gidx0[pl.ds(kb * LANES, LANES)] = v
            else:
                gidx1[pl.ds((kb - KB_HALF) * LANES, LANES)] = v

    def gstart(gidx0, gidx1, rb0, rb1, sem):
        pltpu.async_copy(table_hbm.at[gidx0], rb0, sem)
        pltpu.async_copy(table_hbm.at[gidx1], rb1, sem)

    def gwait(gidx0, gidx1, rb0, rb1, sem):
        pltpu.make_async_copy(table_hbm.at[gidx0], rb0, sem).wait()
        pltpu.make_async_copy(table_hbm.at[gidx1], rb1, sem).wait()

    dvs = [jnp.full((LANES,), d, jnp.int32) for d in range(EMBED_DIM)]

    def transpose(rb0, rb1, tb):
        @plsc.parallel_loop(0, KB_HALF, unroll=4)
        def kb_loop(kb):
            rows = iota + kb * LANES
            for d in range(EMBED_DIM):
                v = plsc.load_gather(rb0, [rows, dvs[d]])
                tb[d, pl.ds(kb * LANES, LANES)] = v
                v = plsc.load_gather(rb1, [rows, dvs[d]])
                tb[d, pl.ds(HALF + kb * LANES, LANES)] = v

    def ostart(h, tb, sem):
        return pltpu.async_copy(
            tb, out_hbm.at[h, :, pl.ds(b0, B_PER_W)], sem)

    def owait(tb, sem):
        pltpu.make_async_copy(
            tb, out_hbm.at[0, :, pl.ds(b0, B_PER_W)], sem).wait()

    regroup(0, gidx_a0, gidx_a1)
    gstart(gidx_a0, gidx_a1, rb_a0, rb_a1, gsem_a)

    @pl.loop(0, HIST, step=2)
    def h_loop(s):
        regroup(s + 1, gidx_b0, gidx_b1)
        gstart(gidx_b0, gidx_b1, rb_b0, rb_b1, gsem_b)
        gwait(gidx_a0, gidx_a1, rb_a0, rb_a1, gsem_a)

        @pl.when(s >= 2)
        def _():
            owait(tb_a, osem_a)
        transpose(rb_a0, rb_a1, tb_a)
        ostart(s, tb_a, osem_a)

        @pl.when(s + 2 < HIST)
        def _():
            regroup(s + 2, gidx_a0, gidx_a1)
            gstart(gidx_a0, gidx_a1, rb_a0, rb_a1, gsem_a)
        gwait(gidx_b0, gidx_b1, rb_b0, rb_b1, gsem_b)

        @pl.when(s >= 2)
        def _():
            owait(tb_b, osem_b)
        transpose(rb_b0, rb_b1, tb_b)
        ostart(s + 1, tb_b, osem_b)

    owait(tb_a, osem_a)
    owait(tb_b, osem_b)


def _embed(idx_flat, weight):
    mesh = plsc.VectorSubcoreMesh(
        core_axis_name="c",
        subcore_axis_name="s",
        num_cores=NUM_CORES,
        num_subcores=NUM_SUBCORES,
    )
    fn = pl.kernel(
        _body,
        out_type=jax.ShapeDtypeStruct((HIST, EMBED_DIM, BATCH), jnp.float32),
        mesh=mesh,
        scratch_types=(
            [pltpu.VMEM((IDX_PER_W,), jnp.int32)]
            + [pltpu.VMEM((HALF,), jnp.int32) for _ in range(4)]
            + [pltpu.VMEM((HALF, EMBED_DIM), jnp.float32) for _ in range(4)]
            + [pltpu.VMEM((EMBED_DIM, B_PER_W), jnp.float32) for _ in range(2)]
            + [pltpu.SemaphoreType.DMA for _ in range(4)]
        ),
        compiler_params=pltpu.CompilerParams(
            use_tc_tiling_on_sc=False, needs_layout_passes=False),
    )
    return fn(idx_flat, weight)


def kernel(input_ids, weight):
    idx_flat = input_ids.reshape(-1).astype(jnp.int32)
    out_t = _embed(idx_flat, weight)
    return jnp.transpose(out_t, (2, 0, 1))


# speedup vs baseline: 1.1057x; 1.1057x over previous
import jax
import jax.numpy as jnp
from jax import lax
from jax.experimental import pallas as pl
from jax.experimental.pallas import tpu as pltpu
from jax.experimental.pallas import tpu_sc as plsc

VOCAB = 1000000
EMBED_DIM = 32
BATCH = 16384
HIST = 50

NUM_CORES = 2
NUM_SUBCORES = 16
NUM_WORKERS = NUM_CORES * NUM_SUBCORES

B_PER_W = BATCH // NUM_WORKERS
IDX_PER_W = B_PER_W * HIST
LANES = 16
KBLOCKS = B_PER_W // LANES


HALF = B_PER_W // 2
KB_HALF = KBLOCKS // 2


def _body(idx_hbm, table_hbm, out_hbm, idx_raw,
          gidx_a0, gidx_a1, gidx_b0, gidx_b1,
          rb_a0, rb_a1, rb_b0, rb_b1, tb_a, tb_b,
          gsem_a, gsem_b, osem_a, osem_b):
    wid = lax.axis_index("s") * NUM_CORES + lax.axis_index("c")
    b0 = wid * B_PER_W

    pltpu.sync_copy(idx_hbm.at[pl.ds(b0 * HIST, IDX_PER_W)], idx_raw)

    iota = lax.iota(jnp.int32, LANES)
    iota_h = iota * HIST

    def regroup(h, gidx0, gidx1):
        for kb in range(KBLOCKS):
            v = plsc.load_gather(idx_raw, [iota_h + (kb * LANES * HIST + h)])
            v = v * 4
            if kb < KB_HALF:
                gidx0[pl.ds(kb * LANES, LANES)] = v
            else:
                gidx1[pl.ds((kb - KB_HALF) * LANES, LANES)] = v

    def gstart(gidx0, gidx1, rb0, rb1, sem):
        pltpu.async_copy(table_hbm.at[gidx0], rb0, sem)
        pltpu.async_copy(table_hbm.at[gidx1], rb1, sem)

    def gwait(gidx0, gidx1, rb0, rb1, sem):
        pltpu.make_async_copy(table_hbm.at[gidx0], rb0, sem).wait()
        pltpu.make_async_copy(table_hbm.at[gidx1], rb1, sem).wait()

    dvs = [jnp.full((LANES,), d, jnp.int32) for d in range(EMBED_DIM)]

    def transpose(rb0, rb1, tb):
        @plsc.parallel_loop(0, KB_HALF, unroll=2)
        def kb_loop(kb):
            rows = iota + kb * LANES
            for d in range(EMBED_DIM):
                v = plsc.load_gather(rb0, [rows, dvs[d]])
                tb[d, pl.ds(kb * LANES, LANES)] = v
                v = plsc.load_gather(rb1, [rows, dvs[d]])
                tb[d, pl.ds(HALF + kb * LANES, LANES)] = v

    def ostart(h, tb, sem):
        return pltpu.async_copy(
            tb, out_hbm.at[h, :, pl.ds(b0, B_PER_W)], sem)

    def owait(tb, sem):
        pltpu.make_async_copy(
            tb, out_hbm.at[0, :, pl.ds(b0, B_PER_W)], sem).wait()

    regroup(0, gidx_a0, gidx_a1)
    gstart(gidx_a0, gidx_a1, rb_a0, rb_a1, gsem_a)

    @pl.loop(0, HIST, step=2)
    def h_loop(s):
        regroup(s + 1, gidx_b0, gidx_b1)
        gstart(gidx_b0, gidx_b1, rb_b0, rb_b1, gsem_b)
        gwait(gidx_a0, gidx_a1, rb_a0, rb_a1, gsem_a)

        @pl.when(s >= 2)
        def _():
            owait(tb_a, osem_a)
        transpose(rb_a0, rb_a1, tb_a)
        ostart(s, tb_a, osem_a)

        @pl.when(s + 2 < HIST)
        def _():
            regroup(s + 2, gidx_a0, gidx_a1)
            gstart(gidx_a0, gidx_a1, rb_a0, rb_a1, gsem_a)
        gwait(gidx_b0, gidx_b1, rb_b0, rb_b1, gsem_b)

        @pl.when(s >= 2)
        def _():
            owait(tb_b, osem_b)
        transpose(rb_b0, rb_b1, tb_b)
        ostart(s + 1, tb_b, osem_b)

    owait(tb_a, osem_a)
    owait(tb_b, osem_b)


def _embed(idx_flat, weight):
    mesh = plsc.VectorSubcoreMesh(
        core_axis_name="c",
        subcore_axis_name="s",
        num_cores=NUM_CORES,
        num_subcores=NUM_SUBCORES,
    )
    fn = pl.kernel(
        _body,
        out_type=jax.ShapeDtypeStruct((HIST, EMBED_DIM, BATCH), jnp.float32),
        mesh=mesh,
        scratch_types=(
            [pltpu.VMEM((IDX_PER_W,), jnp.int32)]
            + [pltpu.VMEM((HALF,), jnp.int32) for _ in range(4)]
            + [pltpu.VMEM((HALF, EMBED_DIM), jnp.float32) for _ in range(4)]
            + [pltpu.VMEM((EMBED_DIM, B_PER_W), jnp.float32) for _ in range(2)]
            + [pltpu.SemaphoreType.DMA for _ in range(4)]
        ),
        compiler_params=pltpu.CompilerParams(
            use_tc_tiling_on_sc=False, needs_layout_passes=False),
    )
    return fn(idx_flat, weight)


def kernel(input_ids, weight):
    idx_flat = input_ids.reshape(-1).astype(jnp.int32)
    w4 = jnp.pad(weight, ((0, 0), (0, 96))).reshape(4 * VOCAB, EMBED_DIM)
    out_t = _embed(idx_flat, w4)
    return jnp.transpose(out_t, (2, 0, 1))
